# baseline (speedup 1.0000x reference)
import functools

import jax
import jax.numpy as jnp
from jax import lax
from jax.experimental import pallas as pl
from jax.experimental.pallas import tpu as pltpu
from jax.experimental.pallas import tpu_sc as plsc

L = 16


def kernel(nodes, neighbors, num_sample, features):
    B, S = neighbors.shape
    N, D = features.shape
    KD = D // L

    info = plsc.get_sparse_core_info()
    NC, NS = info.num_cores, info.num_subcores
    NW = NC * NS

    G = 128 // S
    CH = G * S

    rows_per_w = -(-B // NW)
    rows_per_w = -(-rows_per_w // (4 * G)) * (4 * G)
    nchunk = rows_per_w // G
    nidx = rows_per_w * S

    nb = neighbors.astype(jnp.int32).reshape(B * S)

    feat = features
    n_rows = N
    if n_rows % 8 != 0:
        pad = 8 - n_rows % 8
        feat = jnp.concatenate(
            [feat, jnp.zeros((pad, D), jnp.float32)], axis=0)
        n_rows += pad
    n_stage = -(-(-(-n_rows // NS)) // 8) * 8

    mesh = plsc.VectorSubcoreMesh(core_axis_name="c", subcore_axis_name="s")

    @functools.partial(
        pl.kernel,
        mesh=mesh,
        out_type=jax.ShapeDtypeStruct((B, D), jnp.float32),
        scratch_types=[
            pltpu.VMEM_SHARED((n_rows, D), jnp.float32),
            pltpu.VMEM((nidx,), jnp.int32),
            pltpu.VMEM((CH, D), jnp.float32),
            pltpu.VMEM((CH, D), jnp.float32),
            pltpu.VMEM((2 * G, D), jnp.float32),
            pltpu.VMEM((2 * G, D), jnp.float32),
            pltpu.SemaphoreType.DMA,
            pltpu.SemaphoreType.DMA,
            pltpu.SemaphoreType.DMA,
            pltpu.SemaphoreType.DMA,
        ],
    )
    def sc_kernel(nb_hbm, feat_hbm, out_hbm, tab_sh, idx_v, rows0, rows1,
                  obuf0, obuf1, sem0, sem1, wsem0, wsem1):
        sid = lax.axis_index("s")
        wid = sid * NC + lax.axis_index("c")
        soff = jnp.minimum(sid * n_stage, n_rows - n_stage)
        pltpu.sync_copy(feat_hbm.at[pl.ds(soff, n_stage)],
                        tab_sh.at[pl.ds(soff, n_stage)])
        base = jnp.minimum(wid * rows_per_w, B - rows_per_w)
        pltpu.sync_copy(nb_hbm.at[pl.ds(base * S, nidx)], idx_v)
        plsc.subcore_barrier()

        def start(g, rows, sem):
            pltpu.async_copy(
                tab_sh.at[idx_v.at[pl.ds(g * CH, CH)]], rows, sem)

        def wait_g(rows, sem):
            pltpu.make_async_copy(
                tab_sh.at[idx_v.at[pl.ds(0, CH)]], rows, sem).wait()

        def wait_w(obuf, wsem):
            pltpu.make_async_copy(
                obuf, out_hbm.at[pl.ds(base, 2 * G)], wsem).wait()

        def compute(rows_ref, obuf, half):
            def do_row(r, carry):
                b0 = r * S
                for k in range(KD):
                    sl = pl.ds(k * L, L)
                    acc = [rows_ref[b0 + j, sl] for j in range(4)]
                    for s in range(4, S, 4):
                        for j in range(4):
                            acc[j] = jnp.maximum(
                                acc[j], rows_ref[b0 + s + j, sl])
                    obuf[half * G + r, sl] = jnp.maximum(
                        jnp.maximum(acc[0], acc[1]),
                        jnp.maximum(acc[2], acc[3]))
                return carry

            lax.fori_loop(0, G, do_row, 0)

        start(0, rows0, sem0)
        start(1, rows1, sem1)

        def body(i, carry):
            g = 4 * i
            wait_g(rows0, sem0)

            @pl.when(i > 0)
            def _():
                wait_w(obuf0, wsem0)

            compute(rows0, obuf0, 0)
            start(g + 2, rows0, sem0)

            wait_g(rows1, sem1)
            compute(rows1, obuf0, 1)
            pltpu.async_copy(
                obuf0, out_hbm.at[pl.ds(base + g * G, 2 * G)], wsem0)
            start(g + 3, rows1, sem1)

            wait_g(rows0, sem0)

            @pl.when(i > 0)
            def _():
                wait_w(obuf1, wsem1)

            compute(rows0, obuf1, 0)

            @pl.when(g + 4 < nchunk)
            def _():
                start(g + 4, rows0, sem0)

            wait_g(rows1, sem1)
            compute(rows1, obuf1, 1)
            pltpu.async_copy(
                obuf1, out_hbm.at[pl.ds(base + (g + 2) * G, 2 * G)], wsem1)

            @pl.when(g + 5 < nchunk)
            def _():
                start(g + 5, rows1, sem1)

            return carry

        lax.fori_loop(0, nchunk // 4, body, 0)
        wait_w(obuf0, wsem0)
        wait_w(obuf1, wsem1)

    return sc_kernel(nb, feat)


# speedup vs baseline: 1.4069x; 1.4069x over previous
import functools

import jax
import jax.numpy as jnp
from jax import lax
from jax.experimental import pallas as pl
from jax.experimental.pallas import tpu as pltpu
from jax.experimental.pallas import tpu_sc as plsc

L = 16


def kernel(nodes, neighbors, num_sample, features):
    B, S = neighbors.shape
    N, D = features.shape
    KD = D // L

    info = plsc.get_sparse_core_info()
    NC, NS = info.num_cores, info.num_subcores
    NW = NC * NS

    G = 128 // S
    CH = G * S

    rows_per_w = -(-B // NW)
    rows_per_w = -(-rows_per_w // (4 * G)) * (4 * G)
    nchunk = rows_per_w // G
    nidx = rows_per_w * S

    nb = neighbors.astype(jnp.int32).reshape(B * S)

    feat = features
    n_rows = N
    if n_rows % 8 != 0:
        pad = 8 - n_rows % 8
        feat = jnp.concatenate(
            [feat, jnp.zeros((pad, D), jnp.float32)], axis=0)
        n_rows += pad
    n_stage = -(-(-(-n_rows // NS)) // 8) * 8

    mesh = plsc.VectorSubcoreMesh(core_axis_name="c", subcore_axis_name="s")

    @functools.partial(
        pl.kernel,
        mesh=mesh,
        out_type=jax.ShapeDtypeStruct((B, D), jnp.float32),
        scratch_types=[
            pltpu.VMEM_SHARED((n_rows, D), jnp.float32),
            pltpu.VMEM((nidx,), jnp.int32),
            pltpu.VMEM((CH, D), jnp.float32),
            pltpu.VMEM((CH, D), jnp.float32),
            pltpu.VMEM((2 * G, D), jnp.float32),
            pltpu.VMEM((2 * G, D), jnp.float32),
            pltpu.SemaphoreType.DMA,
            pltpu.SemaphoreType.DMA,
            pltpu.SemaphoreType.DMA,
            pltpu.SemaphoreType.DMA,
        ],
    )
    def sc_kernel(nb_hbm, feat_hbm, out_hbm, tab_sh, idx_v, rows0, rows1,
                  obuf0, obuf1, sem0, sem1, wsem0, wsem1):
        sid = lax.axis_index("s")
        wid = sid * NC + lax.axis_index("c")
        soff = jnp.minimum(sid * n_stage, n_rows - n_stage)
        pltpu.sync_copy(feat_hbm.at[pl.ds(soff, n_stage)],
                        tab_sh.at[pl.ds(soff, n_stage)])
        base = jnp.minimum(wid * rows_per_w, B - rows_per_w)
        pltpu.sync_copy(nb_hbm.at[pl.ds(base * S, nidx)], idx_v)
        plsc.subcore_barrier()

        def start(g, rows, sem):
            pltpu.async_copy(
                tab_sh.at[idx_v.at[pl.ds(g * CH, CH)]], rows, sem)

        def wait_g(rows, sem):
            pltpu.make_async_copy(
                tab_sh.at[idx_v.at[pl.ds(0, CH)]], rows, sem).wait()

        def wait_w(obuf, wsem):
            pltpu.make_async_copy(
                obuf, out_hbm.at[pl.ds(base, 2 * G)], wsem).wait()

        def compute(rows_ref, obuf, half):
            def s_body(s, accs):
                return tuple(
                    jnp.maximum(accs[r * KD + k],
                                rows_ref[r * S + s, pl.ds(k * L, L)])
                    for r in range(G) for k in range(KD))

            accs = tuple(rows_ref[r * S, pl.ds(k * L, L)]
                         for r in range(G) for k in range(KD))
            accs = lax.fori_loop(1, S, s_body, accs)
            for r in range(G):
                for k in range(KD):
                    obuf[half * G + r, pl.ds(k * L, L)] = accs[r * KD + k]

        start(0, rows0, sem0)
        start(1, rows1, sem1)

        def body(i, carry):
            g = 4 * i
            wait_g(rows0, sem0)

            @pl.when(i > 0)
            def _():
                wait_w(obuf0, wsem0)

            compute(rows0, obuf0, 0)
            start(g + 2, rows0, sem0)

            wait_g(rows1, sem1)
            compute(rows1, obuf0, 1)
            pltpu.async_copy(
                obuf0, out_hbm.at[pl.ds(base + g * G, 2 * G)], wsem0)
            start(g + 3, rows1, sem1)

            wait_g(rows0, sem0)

            @pl.when(i > 0)
            def _():
                wait_w(obuf1, wsem1)

            compute(rows0, obuf1, 0)

            @pl.when(g + 4 < nchunk)
            def _():
                start(g + 4, rows0, sem0)

            wait_g(rows1, sem1)
            compute(rows1, obuf1, 1)
            pltpu.async_copy(
                obuf1, out_hbm.at[pl.ds(base + (g + 2) * G, 2 * G)], wsem1)

            @pl.when(g + 5 < nchunk)
            def _():
                start(g + 5, rows1, sem1)

            return carry

        lax.fori_loop(0, nchunk // 4, body, 0)
        wait_w(obuf0, wsem0)
        wait_w(obuf1, wsem1)

    return sc_kernel(nb, feat)


# speedup vs baseline: 1.4209x; 1.0099x over previous
import functools

import jax
import jax.numpy as jnp
from jax import lax
from jax.experimental import pallas as pl
from jax.experimental.pallas import tpu as pltpu
from jax.experimental.pallas import tpu_sc as plsc

L = 16


def kernel(nodes, neighbors, num_sample, features):
    B, S = neighbors.shape
    N, D = features.shape
    KH = D // L

    info = plsc.get_sparse_core_info()
    NC, NS = info.num_cores, info.num_subcores
    NW = NC * NS

    G = 128 // S
    CH = G * S

    rows_per_w = -(-B // NW)
    rows_per_w = -(-rows_per_w // (4 * G)) * (4 * G)
    nchunk = rows_per_w // G
    nidx = rows_per_w * S

    nb = neighbors.astype(jnp.int32).reshape(B * S)

    feat = features
    n_rows = N
    if n_rows % 8 != 0:
        pad = 8 - n_rows % 8
        feat = jnp.concatenate(
            [feat, jnp.zeros((pad, D), jnp.float32)], axis=0)
        n_rows += pad
    n_stage = -(-(-(-n_rows // NS)) // 8) * 8

    mesh = plsc.VectorSubcoreMesh(core_axis_name="c", subcore_axis_name="s")

    @functools.partial(
        pl.kernel,
        mesh=mesh,
        out_type=jax.ShapeDtypeStruct((B, D), jnp.float32),
        scratch_types=[
            pltpu.VMEM_SHARED((n_rows, D), jnp.float32),
            pltpu.VMEM((nidx,), jnp.int32),
            pltpu.VMEM((CH, D), jnp.float32),
            pltpu.VMEM((CH, D), jnp.float32),
            pltpu.VMEM((2 * G, D), jnp.float32),
            pltpu.VMEM((2 * G, D), jnp.float32),
            pltpu.SemaphoreType.DMA,
            pltpu.SemaphoreType.DMA,
            pltpu.SemaphoreType.DMA,
            pltpu.SemaphoreType.DMA,
        ],
    )
    def sc_kernel(nb_hbm, feat_hbm, out_hbm, tab_sh, idx_v, rows0, rows1,
                  obuf0, obuf1, sem0, sem1, wsem0, wsem1):
        sid = lax.axis_index("s")
        wid = sid * NC + lax.axis_index("c")
        soff = jnp.minimum(sid * n_stage, n_rows - n_stage)
        pltpu.sync_copy(feat_hbm.at[pl.ds(soff, n_stage)],
                        tab_sh.at[pl.ds(soff, n_stage)])
        base = jnp.minimum(wid * rows_per_w, B - rows_per_w)
        pltpu.sync_copy(nb_hbm.at[pl.ds(base * S, nidx)], idx_v)
        plsc.subcore_barrier()

        def start(g, rows, sem):
            pltpu.async_copy(
                tab_sh.at[idx_v.at[pl.ds(g * CH, CH)]], rows, sem)

        def wait_g(rows, sem):
            pltpu.make_async_copy(
                tab_sh.at[idx_v.at[pl.ds(0, CH)]], rows, sem).wait()

        def wait_w(obuf, wsem):
            pltpu.make_async_copy(
                obuf, out_hbm.at[pl.ds(base, 2 * G)], wsem).wait()

        def compute(rows_ref, obuf, half):
            def do_row(r, carry):
                b0 = r * S
                accs = tuple(
                    rows_ref[b0, pl.ds(k * L, L)] for k in range(KH))

                def s_body(s, accs):
                    return tuple(
                        jnp.maximum(a, rows_ref[b0 + s, pl.ds(k * L, L)])
                        for k, a in enumerate(accs))

                accs = lax.fori_loop(1, S, s_body, accs)
                for k, a in enumerate(accs):
                    obuf[half * G + r, pl.ds(k * L, L)] = a
                return carry

            lax.fori_loop(0, G, do_row, 0)

        start(0, rows0, sem0)
        start(1, rows1, sem1)

        def body(i, carry):
            g = 4 * i
            wait_g(rows0, sem0)

            @pl.when(i > 0)
            def _():
                wait_w(obuf0, wsem0)

            compute(rows0, obuf0, 0)
            start(g + 2, rows0, sem0)

            wait_g(rows1, sem1)
            compute(rows1, obuf0, 1)
            pltpu.async_copy(
                obuf0, out_hbm.at[pl.ds(base + g * G, 2 * G)], wsem0)
            start(g + 3, rows1, sem1)

            wait_g(rows0, sem0)

            @pl.when(i > 0)
            def _():
                wait_w(obuf1, wsem1)

            compute(rows0, obuf1, 0)

            @pl.when(g + 4 < nchunk)
            def _():
                start(g + 4, rows0, sem0)

            wait_g(rows1, sem1)
            compute(rows1, obuf1, 1)
            pltpu.async_copy(
                obuf1, out_hbm.at[pl.ds(base + (g + 2) * G, 2 * G)], wsem1)

            @pl.when(g + 5 < nchunk)
            def _():
                start(g + 5, rows1, sem1)

            return carry

        lax.fori_loop(0, nchunk // 4, body, 0)
        wait_w(obuf0, wsem0)
        wait_w(obuf1, wsem1)

    return sc_kernel(nb, feat)
